# baseline (speedup 1.0000x reference)
import functools

import jax
import jax.numpy as jnp
from jax import lax
from jax.experimental import pallas as pl
from jax.experimental.pallas import tpu as pltpu
from jax.experimental.pallas import tpu_sc as plsc

B = 16384
D = 32
RPS = 4
SROWS = 250000
NC = 2
NS = 16
L = 16
NW = NC * NS
CHUNK = B // NW
HALF = CHUNK // 2
GSZ = 128
NG = HALF // GSZ

_mesh = plsc.VectorSubcoreMesh(core_axis_name="c", subcore_axis_name="s")


@functools.partial(
    pl.kernel,
    mesh=_mesh,
    out_type=jax.ShapeDtypeStruct((B,), jnp.float32),
    compiler_params=pltpu.CompilerParams(needs_layout_passes=False),
    scratch_types=[
        pltpu.VMEM((HALF,), jnp.int32),
        pltpu.VMEM((HALF,), jnp.int32),
        pltpu.VMEM((HALF,), jnp.int32),
        pltpu.VMEM((HALF,), jnp.int32),
        pltpu.VMEM((HALF, 128), jnp.float32),
        pltpu.VMEM((HALF, 128), jnp.float32),
        pltpu.VMEM((CHUNK,), jnp.float32),
        pltpu.SemaphoreType.DMA,
    ],
)
def _gmf(uidx_hbm, iidx_hbm, utab_hbm, itab_hbm, out_hbm,
         uidx_v, iidx_v, uq_v, iq_v, urows_v, irows_v, out_v, sem):
    wid = lax.axis_index("s") * NC + lax.axis_index("c")
    base = wid * CHUNK

    for p in range(2):
        pbase = base + p * HALF
        pltpu.sync_copy(uidx_hbm.at[pl.ds(pbase, HALF)], uidx_v)
        pltpu.sync_copy(iidx_hbm.at[pl.ds(pbase, HALF)], iidx_v)

        def qbody(i, carry):
            sl = pl.ds(i * L, L)
            uq_v[sl] = lax.shift_right_logical(uidx_v[sl], 2)
            iq_v[sl] = lax.shift_right_logical(iidx_v[sl], 2)
            return carry

        lax.fori_loop(0, HALF // L, qbody, 0)

        copies = []
        for j in range(NG):
            sl = pl.ds(j * GSZ, GSZ)
            copies.append(pltpu.async_copy(utab_hbm.at[uq_v.at[sl]],
                                           urows_v.at[sl], sem))
            copies.append(pltpu.async_copy(itab_hbm.at[iq_v.at[sl]],
                                           irows_v.at[sl], sem))
        for c in copies:
            c.wait()

        def group(g, carry):
            rows = g * L + lax.iota(jnp.int32, L)
            sl = pl.ds(g * L, L)
            uoff = lax.shift_left((uidx_v[sl] & 3), 5)
            ioff = lax.shift_left((iidx_v[sl] & 3), 5)
            acc = jnp.zeros((L,), jnp.float32)
            for d in range(D):
                u = plsc.load_gather(urows_v, [rows, uoff + d])
                w = plsc.load_gather(irows_v, [rows, ioff + d])
                acc = acc + u * w
            out_v[pl.ds(p * HALF + g * L, L)] = acc
            return carry

        lax.fori_loop(0, HALF // L, group, 0)

    pltpu.sync_copy(out_v, out_hbm.at[pl.ds(base, CHUNK)])


def kernel(user_input, item_input, user_table, item_table):
    return _gmf(user_input.astype(jnp.int32), item_input.astype(jnp.int32),
                user_table.reshape(SROWS, 128), item_table.reshape(SROWS, 128))


# speedup vs baseline: 1.0124x; 1.0124x over previous
import functools

import jax
import jax.numpy as jnp
from jax import lax
from jax.experimental import pallas as pl
from jax.experimental.pallas import tpu as pltpu
from jax.experimental.pallas import tpu_sc as plsc

B = 16384
D = 32
NC = 2
NS = 16
L = 16
NW = NC * NS
CHUNK = B // NW
GSZ = 128
NG = CHUNK // GSZ
PACK = 128 // D

_mesh = plsc.VectorSubcoreMesh(core_axis_name="c", subcore_axis_name="s")


@functools.partial(
    pl.kernel,
    mesh=_mesh,
    out_type=jax.ShapeDtypeStruct((B,), jnp.float32),
    compiler_params=pltpu.CompilerParams(needs_layout_passes=False),
    scratch_types=[
        pltpu.VMEM((NG, GSZ), jnp.int32),
        pltpu.VMEM((NG, GSZ), jnp.int32),
        pltpu.VMEM((NG, GSZ), jnp.int32),
        pltpu.VMEM((NG, GSZ), jnp.int32),
        pltpu.VMEM((GSZ, PACK * D), jnp.float32),
        pltpu.VMEM((GSZ, PACK * D), jnp.float32),
        pltpu.VMEM((CHUNK,), jnp.float32),
        pltpu.SemaphoreType.DMA,
    ],
)
def _gmf(uidx_hbm, iidx_hbm, utab_hbm, itab_hbm, out_hbm,
         uraw_v, iraw_v, uq_v, iq_v, ubuf_v, ibuf_v, out_v, sem):
    wid = lax.axis_index("s") * NC + lax.axis_index("c")
    base = wid * CHUNK

    for k in range(NG):
        pltpu.sync_copy(uidx_hbm.at[pl.ds(base + k * GSZ, GSZ)], uraw_v.at[k])
        pltpu.sync_copy(iidx_hbm.at[pl.ds(base + k * GSZ, GSZ)], iraw_v.at[k])

    def shift(j, carry):
        sl = pl.ds(j * L, L)
        uq_v[carry, sl] = lax.shift_right_logical(uraw_v[carry, sl], 2)
        iq_v[carry, sl] = lax.shift_right_logical(iraw_v[carry, sl], 2)
        return carry

    for k in range(NG):
        lax.fori_loop(0, GSZ // L, shift, k)

    lanes = lax.iota(jnp.int32, L)

    def make_group(k):
        def group(g, carry):
            cu_vec = (uraw_v[k, pl.ds(g * L, L)] & (PACK - 1)) * D
            ci_vec = (iraw_v[k, pl.ds(g * L, L)] & (PACK - 1)) * D
            acc = jnp.zeros((L,), jnp.float32)
            for l in range(L):
                r = g * L + l
                rvec = jnp.broadcast_to(r.astype(jnp.int32), (L,))
                cu = cu_vec[l] + lanes
                ci = ci_vec[l] + lanes
                u0 = plsc.load_gather(ubuf_v, [rvec, cu])
                u1 = plsc.load_gather(ubuf_v, [rvec, cu + L])
                i0 = plsc.load_gather(ibuf_v, [rvec, ci])
                i1 = plsc.load_gather(ibuf_v, [rvec, ci + L])
                s = jnp.sum(u0 * i0 + u1 * i1)
                acc = jnp.where(lanes == l, s, acc)
            out_v[pl.ds(k * GSZ + g * L, L)] = acc
            return carry
        return group

    for k in range(NG):
        cu = pltpu.async_copy(utab_hbm.at[uq_v.at[k]], ubuf_v, sem)
        ci = pltpu.async_copy(itab_hbm.at[iq_v.at[k]], ibuf_v, sem)
        cu.wait()
        ci.wait()
        lax.fori_loop(0, GSZ // L, make_group(k), 0)

    pltpu.sync_copy(out_v, out_hbm.at[pl.ds(base, CHUNK)])


def kernel(user_input, item_input, user_table, item_table):
    n_u = user_table.shape[0]
    n_i = item_table.shape[0]
    return _gmf(user_input.astype(jnp.int32), item_input.astype(jnp.int32),
                user_table.reshape(n_u // PACK, PACK * D),
                item_table.reshape(n_i // PACK, PACK * D))


# speedup vs baseline: 1.3946x; 1.3775x over previous
import functools

import jax
import jax.numpy as jnp
from jax import lax
from jax.experimental import pallas as pl
from jax.experimental.pallas import tpu as pltpu
from jax.experimental.pallas import tpu_sc as plsc

B = 16384
D = 32
NC = 2
NS = 16
L = 16
NW = NC * NS
CHUNK = B // NW
PACK = 8
NGRP = CHUNK // L

_mesh = plsc.VectorSubcoreMesh(core_axis_name="c", subcore_axis_name="s")


@functools.partial(
    pl.kernel,
    mesh=_mesh,
    out_type=jax.ShapeDtypeStruct((B,), jnp.float32),
    compiler_params=pltpu.CompilerParams(needs_layout_passes=False),
    scratch_types=[
        pltpu.VMEM((CHUNK,), jnp.int32),
        pltpu.VMEM((CHUNK,), jnp.int32),
        pltpu.VMEM((L, PACK, D), jnp.float32),
        pltpu.VMEM((L, PACK, D), jnp.float32),
        pltpu.VMEM((CHUNK,), jnp.float32),
        pltpu.SemaphoreType.DMA,
    ],
)
def _gmf(uidx_hbm, iidx_hbm, utab_hbm, itab_hbm, out_hbm,
         uidx_v, iidx_v, uslab_v, islab_v, out_v, sem):
    wid = lax.axis_index("s") * NC + lax.axis_index("c")
    base = wid * CHUNK

    pltpu.sync_copy(uidx_hbm.at[pl.ds(base, CHUNK)], uidx_v)
    pltpu.sync_copy(iidx_hbm.at[pl.ds(base, CHUNK)], iidx_v)

    lanes = lax.iota(jnp.int32, L)

    def group(g, carry):
        uvec = uidx_v[pl.ds(g * L, L)]
        ivec = iidx_v[pl.ds(g * L, L)]
        ubase = uvec & ~(PACK - 1)
        ibase = ivec & ~(PACK - 1)
        mu_vec = uvec & (PACK - 1)
        mi_vec = ivec & (PACK - 1)

        copies = []
        for l in range(L):
            bu = pl.multiple_of(ubase[l], PACK)
            bi = pl.multiple_of(ibase[l], PACK)
            copies.append(pltpu.async_copy(
                utab_hbm.at[pl.ds(bu, PACK), :], uslab_v.at[l], sem))
            copies.append(pltpu.async_copy(
                itab_hbm.at[pl.ds(bi, PACK), :], islab_v.at[l], sem))
        for cp in copies:
            cp.wait()

        acc = jnp.zeros((L,), jnp.float32)
        for l in range(L):
            mu = mu_vec[l]
            mi = mi_vec[l]
            u0 = uslab_v[l, mu, pl.ds(0, L)]
            u1 = uslab_v[l, mu, pl.ds(L, L)]
            i0 = islab_v[l, mi, pl.ds(0, L)]
            i1 = islab_v[l, mi, pl.ds(L, L)]
            s = jnp.sum(u0 * i0 + u1 * i1)
            acc = jnp.where(lanes == l, s, acc)
        out_v[pl.ds(g * L, L)] = acc
        return carry

    lax.fori_loop(0, NGRP, group, 0)

    pltpu.sync_copy(out_v, out_hbm.at[pl.ds(base, CHUNK)])


def kernel(user_input, item_input, user_table, item_table):
    return _gmf(user_input.astype(jnp.int32), item_input.astype(jnp.int32),
                user_table, item_table)


# speedup vs baseline: 1.4472x; 1.0378x over previous
import functools

import jax
import jax.numpy as jnp
from jax import lax
from jax.experimental import pallas as pl
from jax.experimental.pallas import tpu as pltpu
from jax.experimental.pallas import tpu_sc as plsc

B = 16384
D = 32
NC = 2
NS = 16
L = 16
NW = NC * NS
CHUNK = B // NW
PACK = 8
NGRP = CHUNK // L
NBUF = 2

_mesh = plsc.VectorSubcoreMesh(core_axis_name="c", subcore_axis_name="s")


@functools.partial(
    pl.kernel,
    mesh=_mesh,
    out_type=jax.ShapeDtypeStruct((B,), jnp.float32),
    compiler_params=pltpu.CompilerParams(needs_layout_passes=False),
    scratch_types=[
        pltpu.VMEM((CHUNK,), jnp.int32),
        pltpu.VMEM((CHUNK,), jnp.int32),
        pltpu.VMEM((NBUF, L * PACK, D), jnp.float32),
        pltpu.VMEM((NBUF, L * PACK, D), jnp.float32),
        pltpu.VMEM((CHUNK,), jnp.float32),
        pltpu.SemaphoreType.DMA,
    ],
)
def _gmf(uidx_hbm, iidx_hbm, utab_hbm, itab_hbm, out_hbm,
         uidx_v, iidx_v, uslab_v, islab_v, out_v, sem):
    wid = lax.axis_index("s") * NC + lax.axis_index("c")
    base = wid * CHUNK

    pltpu.sync_copy(uidx_hbm.at[pl.ds(base, CHUNK)], uidx_v)
    pltpu.sync_copy(iidx_hbm.at[pl.ds(base, CHUNK)], iidx_v)

    lanes = lax.iota(jnp.int32, L)

    def fire(g, slot):
        uvec = uidx_v[pl.ds(g * L, L)]
        ivec = iidx_v[pl.ds(g * L, L)]
        ubase = uvec & ~(PACK - 1)
        ibase = ivec & ~(PACK - 1)
        for l in range(L):
            bu = pl.multiple_of(ubase[l], PACK)
            bi = pl.multiple_of(ibase[l], PACK)
            pltpu.async_copy(utab_hbm.at[pl.ds(bu, PACK), :],
                             uslab_v.at[slot, pl.ds(l * PACK, PACK), :], sem)
            pltpu.async_copy(itab_hbm.at[pl.ds(bi, PACK), :],
                             islab_v.at[slot, pl.ds(l * PACK, PACK), :], sem)

    def drain_compute(g, slot):
        pltpu.make_async_copy(utab_hbm.at[pl.ds(0, L * PACK), :],
                              uslab_v.at[slot], sem).wait()
        pltpu.make_async_copy(itab_hbm.at[pl.ds(0, L * PACK), :],
                              islab_v.at[slot], sem).wait()
        mu_vec = uidx_v[pl.ds(g * L, L)] & (PACK - 1)
        mi_vec = iidx_v[pl.ds(g * L, L)] & (PACK - 1)
        acc = jnp.zeros((L,), jnp.float32)
        for l in range(L):
            ru = l * PACK + mu_vec[l]
            ri = l * PACK + mi_vec[l]
            u0 = uslab_v[slot, ru, pl.ds(0, L)]
            u1 = uslab_v[slot, ru, pl.ds(L, L)]
            i0 = islab_v[slot, ri, pl.ds(0, L)]
            i1 = islab_v[slot, ri, pl.ds(L, L)]
            s = jnp.sum(u0 * i0 + u1 * i1)
            acc = jnp.where(lanes == l, s, acc)
        out_v[pl.ds(g * L, L)] = acc

    for p in range(NBUF):
        fire(jnp.int32(p), p)

    def body(g, carry):
        slot = lax.rem(g, NBUF)
        drain_compute(g, slot)
        nxt = g + NBUF

        @pl.when(nxt < NGRP)
        def _():
            fire(nxt, slot)

        return carry

    lax.fori_loop(0, NGRP, body, 0)

    pltpu.sync_copy(out_v, out_hbm.at[pl.ds(base, CHUNK)])


def kernel(user_input, item_input, user_table, item_table):
    return _gmf(user_input.astype(jnp.int32), item_input.astype(jnp.int32),
                user_table, item_table)
